# baseline (speedup 1.0000x reference)
import functools

import jax
import jax.numpy as jnp
from jax import lax
from jax.experimental import pallas as pl
from jax.experimental.pallas import tpu as pltpu
from jax.experimental.pallas import tpu_sc as plsc

EMB_DIM = 64
BATCH = 4096
SIG_LEN = 50
NUM_IDX = BATCH * SIG_LEN

_NC = 2
_NS = 16
_NW = _NC * _NS
_B_PER_W = NUM_IDX // _NW
_CHUNK = 640
_NCHUNK = _B_PER_W // _CHUNK


def _make_gather():
    mesh = plsc.VectorSubcoreMesh(core_axis_name="c", subcore_axis_name="s")

    @functools.partial(
        pl.kernel,
        mesh=mesh,
        out_type=jax.ShapeDtypeStruct((NUM_IDX, EMB_DIM), jnp.float32),
        scratch_types=[
            pltpu.VMEM((_B_PER_W,), jnp.int32),
            pltpu.VMEM((_CHUNK, EMB_DIM), jnp.float32),
            pltpu.SemaphoreType.DMA,
        ],
        compiler_params=pltpu.CompilerParams(use_tc_tiling_on_sc=False),
    )
    def gather_kernel(idx_hbm, table_hbm, out_hbm, idx_v, rows_v, sem):
        wid = lax.axis_index("s") * _NC + lax.axis_index("c")
        base = pl.multiple_of(wid * _B_PER_W, _B_PER_W)
        pltpu.sync_copy(idx_hbm.at[pl.ds(base, _B_PER_W)], idx_v)
        for j in range(_NCHUNK):
            off = j * _CHUNK
            pltpu.async_copy(
                table_hbm.at[idx_v.at[pl.ds(off, _CHUNK)]], rows_v, sem
            ).wait()
            pltpu.sync_copy(rows_v, out_hbm.at[pl.ds(base + off, _CHUNK)])

    return gather_kernel


_gather = _make_gather()


@jax.jit
def kernel(news_batch, table):
    idx = news_batch.reshape(NUM_IDX).astype(jnp.int32)
    out = _gather(idx, table)
    return out.reshape(BATCH, SIG_LEN, EMB_DIM)


# speedup vs baseline: 1.0097x; 1.0097x over previous
import functools

import jax
import jax.numpy as jnp
from jax import lax
from jax.experimental import pallas as pl
from jax.experimental.pallas import tpu as pltpu
from jax.experimental.pallas import tpu_sc as plsc

EMB_DIM = 64
BATCH = 4096
SIG_LEN = 50
NUM_IDX = BATCH * SIG_LEN

_NC = 2
_NS = 16
_NW = _NC * _NS
_B_PER_W = NUM_IDX // _NW
_CHUNK = 400
_NCHUNK = _B_PER_W // _CHUNK
_NBUF = 4


def _make_gather():
    mesh = plsc.VectorSubcoreMesh(core_axis_name="c", subcore_axis_name="s")

    @functools.partial(
        pl.kernel,
        mesh=mesh,
        out_type=jax.ShapeDtypeStruct((NUM_IDX, EMB_DIM), jnp.float32),
        scratch_types=[
            pltpu.VMEM((_B_PER_W,), jnp.int32),
            pltpu.VMEM((_NBUF, _CHUNK, EMB_DIM), jnp.float32),
            pltpu.SemaphoreType.DMA((_NBUF,)),
            pltpu.SemaphoreType.DMA((_NBUF,)),
        ],
        compiler_params=pltpu.CompilerParams(use_tc_tiling_on_sc=False),
    )
    def gather_kernel(idx_hbm, table_hbm, out_hbm, idx_v, rows_v, gsem, osem):
        wid = lax.axis_index("s") * _NC + lax.axis_index("c")
        base = pl.multiple_of(wid * _B_PER_W, _B_PER_W)
        pltpu.sync_copy(idx_hbm.at[pl.ds(base, _B_PER_W)], idx_v)

        def gather(j):
            b = j % _NBUF
            return pltpu.async_copy(
                table_hbm.at[idx_v.at[pl.ds(j * _CHUNK, _CHUNK)]],
                rows_v.at[b],
                gsem.at[b],
            )

        def writeback(j):
            b = j % _NBUF
            return pltpu.make_async_copy(
                rows_v.at[b], out_hbm.at[pl.ds(base + j * _CHUNK, _CHUNK)],
                osem.at[b],
            )

        gathers = [gather(j) for j in range(_NBUF - 1)]
        writebacks = [None] * _NCHUNK
        for j in range(_NCHUNK):
            gathers[j].wait()
            writebacks[j] = writeback(j)
            writebacks[j].start()
            k = j + _NBUF - 1
            if k < _NCHUNK:
                if k >= _NBUF:
                    writebacks[k - _NBUF].wait()
                gathers.append(gather(k))
        for j in range(_NCHUNK - _NBUF, _NCHUNK):
            writebacks[j].wait()

    return gather_kernel


_gather = _make_gather()


@jax.jit
def kernel(news_batch, table):
    idx = news_batch.reshape(NUM_IDX).astype(jnp.int32)
    out = _gather(idx, table)
    return out.reshape(BATCH, SIG_LEN, EMB_DIM)
